# baseline (speedup 1.0000x reference)
import jax
import jax.numpy as jnp
from jax import lax
from jax.experimental import pallas as pl
from jax.experimental.pallas import tpu as pltpu
from jax.experimental.pallas import tpu_sc as plsc

N_NODES = 1_000_000
D = 64
B = 16384
NC = 2
NS = 16
NW = NC * NS
BPW = B // NW
CH = 128
NCH = BPW // CH
ROWS2 = B // CH
R = ROWS2 // NS
TRASH = N_NODES
K_ROUNDS = 4
LANES = 16
_TAG_DISABLE = 16


def _mesh():
    return plsc.VectorSubcoreMesh(core_axis_name="c", subcore_axis_name="s")


def _gather_tag_body(mem_hbm, ids2_hbm, biota_hbm, h_hbm, t2_hbm, tag_hbm,
                     idx_v, rows_v, tidx_v, biota_v, tvals_v, sidx_v,
                     sem, sem2):
    c = lax.axis_index("c")
    s = lax.axis_index("s")
    wid = s * NC + c

    pltpu.sync_copy(ids2_hbm.at[pl.ds(wid * NCH, NCH)], idx_v)
    for ch in range(NCH):
        pltpu.async_copy(mem_hbm.at[idx_v.at[ch]],
                         rows_v.at[pl.ds(ch * CH, CH)], sem).wait()
    pltpu.sync_copy(rows_v, h_hbm.at[pl.ds(wid * BPW, BPW)])

    @pl.when((c == 0) & (s < _TAG_DISABLE))
    def _():
        pltpu.sync_copy(ids2_hbm.at[pl.ds(s * R, R)], tidx_v)
        pltpu.sync_copy(biota_hbm.at[pl.ds(s * R, R)], biota_v)
        for r in range(R):
            pltpu.sync_copy(biota_v.at[r], tag_hbm.at[tidx_v.at[r]])
        for _k in range(K_ROUNDS):
            plsc.subcore_barrier()
            for r in range(R):
                pltpu.async_copy(tag_hbm.at[tidx_v.at[r]], tvals_v.at[r],
                                 sem2).wait()
            for r in range(R):
                for j in range(CH // LANES):
                    sl = pl.ds(j * LANES, LANES)
                    tv = tvals_v[r, sl]
                    bv = biota_v[r, sl]
                    iv = tidx_v[r, sl]
                    sidx_v[r, sl] = jnp.where(bv > tv, iv, TRASH + bv)
            plsc.subcore_barrier()
            for r in range(R):
                pltpu.sync_copy(biota_v.at[r], tag_hbm.at[sidx_v.at[r]])
        plsc.subcore_barrier()
        for r in range(R):
            pltpu.async_copy(tag_hbm.at[tidx_v.at[r]], tvals_v.at[r],
                             sem2).wait()
        pltpu.sync_copy(tvals_v, t2_hbm.at[pl.ds(s * R, R)])


_gather_and_tag = pl.kernel(
    _gather_tag_body,
    out_type=(
        jax.ShapeDtypeStruct((B, D), jnp.float32),
        jax.ShapeDtypeStruct((ROWS2, CH), jnp.int32),
        jax.ShapeDtypeStruct((N_NODES + B,), jnp.int32),
    ),
    mesh=_mesh(),
    scratch_types=[
        pltpu.VMEM((NCH, CH), jnp.int32),
        pltpu.VMEM((BPW, D), jnp.float32),
        pltpu.VMEM((R, CH), jnp.int32),
        pltpu.VMEM((R, CH), jnp.int32),
        pltpu.VMEM((R, CH), jnp.int32),
        pltpu.VMEM((R, CH), jnp.int32),
        pltpu.SemaphoreType.DMA,
        pltpu.SemaphoreType.DMA,
    ],
    compiler_params=pltpu.CompilerParams(use_tc_tiling_on_sc=False),
)


GRU_BLK = 1024


def _gru_body(x_ref, h_ref, wih_ref, whh_ref, bih_ref, bhh_ref, o_ref):
    x = x_ref[...]
    h = h_ref[...]
    gi = jnp.dot(x, wih_ref[...], preferred_element_type=jnp.float32) + bih_ref[...]
    gh = jnp.dot(h, whh_ref[...], preferred_element_type=jnp.float32) + bhh_ref[...]
    r = jax.nn.sigmoid(gi[:, :D] + gh[:, :D])
    z = jax.nn.sigmoid(gi[:, D:2 * D] + gh[:, D:2 * D])
    n = jnp.tanh(gi[:, 2 * D:] + r * gh[:, 2 * D:])
    o_ref[...] = (1.0 - z) * n + z * h


def _gru(msgs, h, w_ih_t, w_hh_t, b_ih2, b_hh2):
    return pl.pallas_call(
        _gru_body,
        grid=(B // GRU_BLK,),
        in_specs=[
            pl.BlockSpec((GRU_BLK, D), lambda i: (i, 0)),
            pl.BlockSpec((GRU_BLK, D), lambda i: (i, 0)),
            pl.BlockSpec((D, 3 * D), lambda i: (0, 0)),
            pl.BlockSpec((D, 3 * D), lambda i: (0, 0)),
            pl.BlockSpec((1, 3 * D), lambda i: (0, 0)),
            pl.BlockSpec((1, 3 * D), lambda i: (0, 0)),
        ],
        out_specs=pl.BlockSpec((GRU_BLK, D), lambda i: (i, 0)),
        out_shape=jax.ShapeDtypeStruct((B, D), jnp.float32),
    )(msgs, h, w_ih_t, w_hh_t, b_ih2, b_hh2)


def _scatter_body(newh_hbm, t2_hbm, ids2_hbm, ts_hbm, mem_ref, tim_ref,
                  idx_v, tw_v, rows_v, tsr_v, sem):
    c = lax.axis_index("c")
    s = lax.axis_index("s")
    wid = s * NC + c
    pltpu.sync_copy(ids2_hbm.at[pl.ds(wid * NCH, NCH)], idx_v)
    pltpu.sync_copy(t2_hbm.at[pl.ds(wid * NCH, NCH)], tw_v)
    for ch in range(NCH):
        pltpu.async_copy(newh_hbm.at[tw_v.at[ch]],
                         rows_v.at[pl.ds(ch * CH, CH)], sem).wait()
        pltpu.async_copy(ts_hbm.at[tw_v.at[ch]], tsr_v.at[ch], sem).wait()
        pltpu.sync_copy(rows_v.at[pl.ds(ch * CH, CH)], mem_ref.at[idx_v.at[ch]])
        pltpu.sync_copy(tsr_v.at[ch], tim_ref.at[idx_v.at[ch]])


_scatter = pl.kernel(
    _scatter_body,
    out_type=(),
    mesh=_mesh(),
    scratch_types=[
        pltpu.VMEM((NCH, CH), jnp.int32),
        pltpu.VMEM((NCH, CH), jnp.int32),
        pltpu.VMEM((BPW, D), jnp.float32),
        pltpu.VMEM((NCH, CH), jnp.float32),
        pltpu.SemaphoreType.DMA,
    ],
    compiler_params=pltpu.CompilerParams(use_tc_tiling_on_sc=False),
)


def kernel(node_memories, node_last_updated_times, unique_node_ids,
           unique_node_messages, unique_node_timestamps,
           W_ih, W_hh, b_ih, b_hh):
    ids2 = unique_node_ids.reshape(ROWS2, CH)
    biota = jnp.arange(B, dtype=jnp.int32).reshape(ROWS2, CH)
    h, t2, _tag = _gather_and_tag(node_memories, ids2, biota)
    if _TAG_DISABLE == 0:
        t2 = biota
    new_h = _gru(unique_node_messages, h, W_ih.T, W_hh.T,
                 b_ih.reshape(1, 3 * D), b_hh.reshape(1, 3 * D))
    mem_ref = jax.new_ref(node_memories)
    tim_ref = jax.new_ref(node_last_updated_times)
    _scatter(new_h, t2, ids2, unique_node_timestamps, mem_ref, tim_ref)
    return jax.freeze(mem_ref), jax.freeze(tim_ref)


# speedup vs baseline: 1.0101x; 1.0101x over previous
import jax
import jax.numpy as jnp
from jax import lax
from jax.experimental import pallas as pl
from jax.experimental.pallas import tpu as pltpu
from jax.experimental.pallas import tpu_sc as plsc

N_NODES = 1_000_000
D = 64
B = 16384
NC = 2
NS = 16
NW = NC * NS
BPW = B // NW
CH = 128
NCH = BPW // CH
ROWS2 = B // CH
R = ROWS2 // NS
TRASH = N_NODES
K_ROUNDS = 4
LANES = 16


def _mesh():
    return plsc.VectorSubcoreMesh(core_axis_name="c", subcore_axis_name="s")


def _gather_tag_body(mem_hbm, ids2_hbm, biota_hbm, h_hbm, t2_hbm, tag_hbm,
                     idx_v, rows_v, tidx_v, biota_v, tvals_v, sidx_v,
                     sem, sem2):
    c = lax.axis_index("c")
    s = lax.axis_index("s")
    wid = s * NC + c

    pltpu.sync_copy(ids2_hbm.at[pl.ds(wid * NCH, NCH)], idx_v)
    gathers = [
        pltpu.async_copy(mem_hbm.at[idx_v.at[ch]],
                         rows_v.at[pl.ds(ch * CH, CH)], sem)
        for ch in range(NCH)
    ]
    for g in gathers:
        g.wait()
    pltpu.sync_copy(rows_v, h_hbm.at[pl.ds(wid * BPW, BPW)])

    @pl.when(c == 0)
    def _():
        pltpu.sync_copy(ids2_hbm.at[pl.ds(s * R, R)], tidx_v)
        pltpu.sync_copy(biota_hbm.at[pl.ds(s * R, R)], biota_v)
        ts0 = [pltpu.async_copy(biota_v.at[r], tag_hbm.at[tidx_v.at[r]], sem2)
               for r in range(R)]
        for g in ts0:
            g.wait()
        for _k in range(K_ROUNDS):
            plsc.subcore_barrier()
            tg = [pltpu.async_copy(tag_hbm.at[tidx_v.at[r]], tvals_v.at[r],
                                   sem2) for r in range(R)]
            for g in tg:
                g.wait()
            for r in range(R):
                for j in range(CH // LANES):
                    sl = pl.ds(j * LANES, LANES)
                    tv = tvals_v[r, sl]
                    bv = biota_v[r, sl]
                    iv = tidx_v[r, sl]
                    sidx_v[r, sl] = jnp.where(bv > tv, iv, TRASH + bv)
            plsc.subcore_barrier()
            tsc = [pltpu.async_copy(biota_v.at[r], tag_hbm.at[sidx_v.at[r]],
                                    sem2) for r in range(R)]
            for g in tsc:
                g.wait()
        plsc.subcore_barrier()
        tg = [pltpu.async_copy(tag_hbm.at[tidx_v.at[r]], tvals_v.at[r], sem2)
              for r in range(R)]
        for g in tg:
            g.wait()
        pltpu.sync_copy(tvals_v, t2_hbm.at[pl.ds(s * R, R)])


_gather_and_tag = pl.kernel(
    _gather_tag_body,
    out_type=(
        jax.ShapeDtypeStruct((B, D), jnp.float32),
        jax.ShapeDtypeStruct((ROWS2, CH), jnp.int32),
        jax.ShapeDtypeStruct((N_NODES + B,), jnp.int32),
    ),
    mesh=_mesh(),
    scratch_types=[
        pltpu.VMEM((NCH, CH), jnp.int32),
        pltpu.VMEM((BPW, D), jnp.float32),
        pltpu.VMEM((R, CH), jnp.int32),
        pltpu.VMEM((R, CH), jnp.int32),
        pltpu.VMEM((R, CH), jnp.int32),
        pltpu.VMEM((R, CH), jnp.int32),
        pltpu.SemaphoreType.DMA,
        pltpu.SemaphoreType.DMA,
    ],
    compiler_params=pltpu.CompilerParams(use_tc_tiling_on_sc=False),
)


GRU_BLK = 1024


def _gru_body(x_ref, h_ref, wih_ref, whh_ref, bih_ref, bhh_ref, o_ref):
    x = x_ref[...]
    h = h_ref[...]
    gi = jnp.dot(x, wih_ref[...], preferred_element_type=jnp.float32) + bih_ref[...]
    gh = jnp.dot(h, whh_ref[...], preferred_element_type=jnp.float32) + bhh_ref[...]
    r = jax.nn.sigmoid(gi[:, :D] + gh[:, :D])
    z = jax.nn.sigmoid(gi[:, D:2 * D] + gh[:, D:2 * D])
    n = jnp.tanh(gi[:, 2 * D:] + r * gh[:, 2 * D:])
    o_ref[...] = (1.0 - z) * n + z * h


def _gru(msgs, h, w_ih_t, w_hh_t, b_ih2, b_hh2):
    return pl.pallas_call(
        _gru_body,
        grid=(B // GRU_BLK,),
        in_specs=[
            pl.BlockSpec((GRU_BLK, D), lambda i: (i, 0)),
            pl.BlockSpec((GRU_BLK, D), lambda i: (i, 0)),
            pl.BlockSpec((D, 3 * D), lambda i: (0, 0)),
            pl.BlockSpec((D, 3 * D), lambda i: (0, 0)),
            pl.BlockSpec((1, 3 * D), lambda i: (0, 0)),
            pl.BlockSpec((1, 3 * D), lambda i: (0, 0)),
        ],
        out_specs=pl.BlockSpec((GRU_BLK, D), lambda i: (i, 0)),
        out_shape=jax.ShapeDtypeStruct((B, D), jnp.float32),
    )(msgs, h, w_ih_t, w_hh_t, b_ih2, b_hh2)


def _scatter_body(newh_hbm, t2_hbm, ids2_hbm, ts_hbm, mem_ref, tim_ref,
                  idx_v, tw_v, rows_v, tsr_v, sem):
    c = lax.axis_index("c")
    s = lax.axis_index("s")
    wid = s * NC + c
    pltpu.sync_copy(ids2_hbm.at[pl.ds(wid * NCH, NCH)], idx_v)
    pltpu.sync_copy(t2_hbm.at[pl.ds(wid * NCH, NCH)], tw_v)
    for ch in range(NCH):
        pltpu.async_copy(newh_hbm.at[tw_v.at[ch]],
                         rows_v.at[pl.ds(ch * CH, CH)], sem).wait()
        pltpu.async_copy(ts_hbm.at[tw_v.at[ch]], tsr_v.at[ch], sem).wait()
        pltpu.sync_copy(rows_v.at[pl.ds(ch * CH, CH)], mem_ref.at[idx_v.at[ch]])
        pltpu.sync_copy(tsr_v.at[ch], tim_ref.at[idx_v.at[ch]])


_scatter = pl.kernel(
    _scatter_body,
    out_type=(),
    mesh=_mesh(),
    scratch_types=[
        pltpu.VMEM((NCH, CH), jnp.int32),
        pltpu.VMEM((NCH, CH), jnp.int32),
        pltpu.VMEM((BPW, D), jnp.float32),
        pltpu.VMEM((NCH, CH), jnp.float32),
        pltpu.SemaphoreType.DMA,
    ],
    compiler_params=pltpu.CompilerParams(use_tc_tiling_on_sc=False),
)


def kernel(node_memories, node_last_updated_times, unique_node_ids,
           unique_node_messages, unique_node_timestamps,
           W_ih, W_hh, b_ih, b_hh):
    ids2 = unique_node_ids.reshape(ROWS2, CH)
    biota = jnp.arange(B, dtype=jnp.int32).reshape(ROWS2, CH)
    mem_ref = jax.new_ref(node_memories)
    tim_ref = jax.new_ref(node_last_updated_times)
    h, t2, _tag = _gather_and_tag(mem_ref, ids2, biota)
    new_h = _gru(unique_node_messages, h, W_ih.T, W_hh.T,
                 b_ih.reshape(1, 3 * D), b_hh.reshape(1, 3 * D))
    _scatter(new_h, t2, ids2, unique_node_timestamps, mem_ref, tim_ref)
    return jax.freeze(mem_ref), jax.freeze(tim_ref)


# speedup vs baseline: 1.4447x; 1.4302x over previous
import jax
import jax.numpy as jnp
from jax import lax
from jax.experimental import pallas as pl
from jax.experimental.pallas import tpu as pltpu
from jax.experimental.pallas import tpu_sc as plsc

N_NODES = 1_000_000
D = 64
B = 16384
NC = 2
NS = 16
NW = NC * NS
BPW = B // NW
CH = 128
NCH = BPW // CH
ROWS2 = B // CH
R = ROWS2 // NS
TRASH = N_NODES
K_ROUNDS = 4
LANES = 16
KF = 24


def _mesh():
    return plsc.VectorSubcoreMesh(core_axis_name="c", subcore_axis_name="s")


def _scalar_at(vref, k):
    v = vref[pl.ds((k // LANES) * LANES, LANES)]
    lane = lax.broadcasted_iota(jnp.int32, (LANES,), 0)
    return jnp.max(jnp.where(lane == (k % LANES), v, jnp.int32(-1)))


def _tag_body(ids2_hbm, biota_hbm, ts_hbm, tim_ref, t2_hbm, tag_hbm,
              tidx_v, biota_v, tvals_v, sidx_v, tsr_v, sem2):
    c = lax.axis_index("c")
    s = lax.axis_index("s")

    @pl.when(c == 0)
    def _():
        pltpu.sync_copy(ids2_hbm.at[pl.ds(s * R, R)], tidx_v)
        pltpu.sync_copy(biota_hbm.at[pl.ds(s * R, R)], biota_v)
        ts0 = [pltpu.async_copy(biota_v.at[r], tag_hbm.at[tidx_v.at[r]], sem2)
               for r in range(R)]
        for g in ts0:
            g.wait()
        for _k in range(K_ROUNDS):
            plsc.subcore_barrier()
            tg = [pltpu.async_copy(tag_hbm.at[tidx_v.at[r]], tvals_v.at[r],
                                   sem2) for r in range(R)]
            for g in tg:
                g.wait()
            for r in range(R):
                for j in range(CH // LANES):
                    sl = pl.ds(j * LANES, LANES)
                    tv = tvals_v[r, sl]
                    bv = biota_v[r, sl]
                    iv = tidx_v[r, sl]
                    sidx_v[r, sl] = jnp.where(bv > tv, iv, TRASH + bv)
            plsc.subcore_barrier()
            tsc = [pltpu.async_copy(biota_v.at[r], tag_hbm.at[sidx_v.at[r]],
                                    sem2) for r in range(R)]
            for g in tsc:
                g.wait()
        plsc.subcore_barrier()
        tg = [pltpu.async_copy(tag_hbm.at[tidx_v.at[r]], tvals_v.at[r], sem2)
              for r in range(R)]
        for g in tg:
            g.wait()
        pltpu.sync_copy(tvals_v, t2_hbm.at[pl.ds(s * R, R)])
        tsg = [pltpu.async_copy(ts_hbm.at[tvals_v.at[r]], tsr_v.at[r], sem2)
               for r in range(R)]
        for g in tsg:
            g.wait()
        tss = [pltpu.async_copy(tsr_v.at[r], tim_ref.at[tidx_v.at[r]], sem2)
               for r in range(R)]
        for g in tss:
            g.wait()


_tag_and_times = pl.kernel(
    _tag_body,
    out_type=(
        jax.ShapeDtypeStruct((ROWS2, CH), jnp.int32),
        jax.ShapeDtypeStruct((N_NODES + B,), jnp.int32),
    ),
    mesh=_mesh(),
    scratch_types=[
        pltpu.VMEM((R, CH), jnp.int32),
        pltpu.VMEM((R, CH), jnp.int32),
        pltpu.VMEM((R, CH), jnp.int32),
        pltpu.VMEM((R, CH), jnp.int32),
        pltpu.VMEM((R, CH), jnp.float32),
        pltpu.SemaphoreType.DMA,
    ],
    compiler_params=pltpu.CompilerParams(use_tc_tiling_on_sc=False),
)


def _gather_body(mem_ref, ids_hbm, h_hbm, idx_v, rows_v, sem):
    c = lax.axis_index("c")
    s = lax.axis_index("s")
    wid = s * NC + c
    base = wid * BPW
    pltpu.sync_copy(ids_hbm.at[pl.ds(base, BPW)], idx_v)

    @pl.loop(0, BPW)
    def _fire(k):
        i = _scalar_at(idx_v, k)
        pltpu.async_copy(mem_ref.at[pl.ds(i, 1)], rows_v.at[pl.ds(k, 1)], sem)

        @pl.when(k >= KF)
        def _():
            pltpu.make_async_copy(mem_ref.at[pl.ds(0, 1)],
                                  rows_v.at[pl.ds(k - KF, 1)], sem).wait()

    @pl.loop(BPW - KF, BPW)
    def _drain(k):
        pltpu.make_async_copy(mem_ref.at[pl.ds(0, 1)],
                              rows_v.at[pl.ds(k, 1)], sem).wait()

    pltpu.sync_copy(rows_v, h_hbm.at[pl.ds(base, BPW)])


_gather = pl.kernel(
    _gather_body,
    out_type=jax.ShapeDtypeStruct((B, D), jnp.float32),
    mesh=_mesh(),
    scratch_types=[
        pltpu.VMEM((BPW,), jnp.int32),
        pltpu.VMEM((BPW, D), jnp.float32),
        pltpu.SemaphoreType.DMA,
    ],
    compiler_params=pltpu.CompilerParams(needs_layout_passes=False),
)


GRU_BLK = 1024


def _gru_body(x_ref, h_ref, wih_ref, whh_ref, bih_ref, bhh_ref, o_ref):
    x = x_ref[...]
    h = h_ref[...]
    gi = jnp.dot(x, wih_ref[...], preferred_element_type=jnp.float32) + bih_ref[...]
    gh = jnp.dot(h, whh_ref[...], preferred_element_type=jnp.float32) + bhh_ref[...]
    r = jax.nn.sigmoid(gi[:, :D] + gh[:, :D])
    z = jax.nn.sigmoid(gi[:, D:2 * D] + gh[:, D:2 * D])
    n = jnp.tanh(gi[:, 2 * D:] + r * gh[:, 2 * D:])
    o_ref[...] = (1.0 - z) * n + z * h


def _gru(msgs, h, w_ih_t, w_hh_t, b_ih2, b_hh2):
    return pl.pallas_call(
        _gru_body,
        grid=(B // GRU_BLK,),
        in_specs=[
            pl.BlockSpec((GRU_BLK, D), lambda i: (i, 0)),
            pl.BlockSpec((GRU_BLK, D), lambda i: (i, 0)),
            pl.BlockSpec((D, 3 * D), lambda i: (0, 0)),
            pl.BlockSpec((D, 3 * D), lambda i: (0, 0)),
            pl.BlockSpec((1, 3 * D), lambda i: (0, 0)),
            pl.BlockSpec((1, 3 * D), lambda i: (0, 0)),
        ],
        out_specs=pl.BlockSpec((GRU_BLK, D), lambda i: (i, 0)),
        out_shape=jax.ShapeDtypeStruct((B, D), jnp.float32),
    )(msgs, h, w_ih_t, w_hh_t, b_ih2, b_hh2)


def _scatter_body(newh_hbm, t2_hbm, ids_hbm, mem_ref,
                  idx_v, tw_v, rows_v, sem, sem2):
    c = lax.axis_index("c")
    s = lax.axis_index("s")
    wid = s * NC + c
    base = wid * BPW
    pltpu.sync_copy(ids_hbm.at[pl.ds(base, BPW)], idx_v)
    pltpu.sync_copy(t2_hbm.at[pl.ds(base, BPW)], tw_v)

    @pl.loop(0, BPW)
    def _fire_g(k):
        t = _scalar_at(tw_v, k)
        pltpu.async_copy(newh_hbm.at[pl.ds(t, 1)], rows_v.at[pl.ds(k, 1)], sem)

        @pl.when(k >= KF)
        def _():
            pltpu.make_async_copy(newh_hbm.at[pl.ds(0, 1)],
                                  rows_v.at[pl.ds(k - KF, 1)], sem).wait()

    @pl.loop(BPW - KF, BPW)
    def _drain_g(k):
        pltpu.make_async_copy(newh_hbm.at[pl.ds(0, 1)],
                              rows_v.at[pl.ds(k, 1)], sem).wait()

    @pl.loop(0, BPW)
    def _fire_s(k):
        i = _scalar_at(idx_v, k)
        pltpu.async_copy(rows_v.at[pl.ds(k, 1)], mem_ref.at[pl.ds(i, 1)], sem2)

        @pl.when(k >= KF)
        def _():
            pltpu.make_async_copy(rows_v.at[pl.ds(k - KF, 1)],
                                  mem_ref.at[pl.ds(0, 1)], sem2).wait()

    @pl.loop(BPW - KF, BPW)
    def _drain_s(k):
        pltpu.make_async_copy(rows_v.at[pl.ds(k, 1)],
                              mem_ref.at[pl.ds(0, 1)], sem2).wait()


_scatter = pl.kernel(
    _scatter_body,
    out_type=(),
    mesh=_mesh(),
    scratch_types=[
        pltpu.VMEM((BPW,), jnp.int32),
        pltpu.VMEM((BPW,), jnp.int32),
        pltpu.VMEM((BPW, D), jnp.float32),
        pltpu.SemaphoreType.DMA,
        pltpu.SemaphoreType.DMA,
    ],
    compiler_params=pltpu.CompilerParams(needs_layout_passes=False),
)


def kernel(node_memories, node_last_updated_times, unique_node_ids,
           unique_node_messages, unique_node_timestamps,
           W_ih, W_hh, b_ih, b_hh):
    ids2 = unique_node_ids.reshape(ROWS2, CH)
    biota = jnp.arange(B, dtype=jnp.int32).reshape(ROWS2, CH)
    mem_ref = jax.new_ref(node_memories)
    tim_ref = jax.new_ref(node_last_updated_times)
    t2, _tag = _tag_and_times(ids2, biota, unique_node_timestamps, tim_ref)
    h = _gather(mem_ref, unique_node_ids)
    new_h = _gru(unique_node_messages, h, W_ih.T, W_hh.T,
                 b_ih.reshape(1, 3 * D), b_hh.reshape(1, 3 * D))
    _scatter(new_h, t2.reshape(B), unique_node_ids, mem_ref)
    return jax.freeze(mem_ref), jax.freeze(tim_ref)


# speedup vs baseline: 1.8085x; 1.2519x over previous
import jax
import jax.numpy as jnp
from jax import lax
from jax.experimental import pallas as pl
from jax.experimental.pallas import tpu as pltpu
from jax.experimental.pallas import tpu_sc as plsc

N_NODES = 1_000_000
D = 64
B = 16384
NC = 2
NS = 16
NW = NC * NS
BPW = B // NW
CH = 128
NCH = BPW // CH
ROWS2 = B // CH
R = ROWS2 // NS
TRASH = N_NODES
K_ROUNDS = 4
LANES = 16
KF = 24


def _mesh():
    return plsc.VectorSubcoreMesh(core_axis_name="c", subcore_axis_name="s")


def _scalar_at(vref, k):
    v = vref[pl.ds((k // LANES) * LANES, LANES)]
    lane = lax.broadcasted_iota(jnp.int32, (LANES,), 0)
    return jnp.max(jnp.where(lane == (k % LANES), v, jnp.int32(-1)))


def _tag_body(ids2_hbm, biota_hbm, ts_hbm, tim_ref, t2_hbm, tag_hbm,
              tidx_v, biota_v, tvals_v, sidx_v, tsr_v, sem2):
    c = lax.axis_index("c")
    s = lax.axis_index("s")

    @pl.when(c == 0)
    def _():
        pltpu.sync_copy(ids2_hbm.at[pl.ds(s * R, R)], tidx_v)
        pltpu.sync_copy(biota_hbm.at[pl.ds(s * R, R)], biota_v)
        ts0 = [pltpu.async_copy(biota_v.at[r], tag_hbm.at[tidx_v.at[r]], sem2)
               for r in range(R)]
        for g in ts0:
            g.wait()
        for _k in range(K_ROUNDS):
            plsc.subcore_barrier()
            tg = [pltpu.async_copy(tag_hbm.at[tidx_v.at[r]], tvals_v.at[r],
                                   sem2) for r in range(R)]
            for g in tg:
                g.wait()
            for r in range(R):
                for j in range(CH // LANES):
                    sl = pl.ds(j * LANES, LANES)
                    tv = tvals_v[r, sl]
                    bv = biota_v[r, sl]
                    iv = tidx_v[r, sl]
                    sidx_v[r, sl] = jnp.where(bv > tv, iv, TRASH + bv)
            plsc.subcore_barrier()
            tsc = [pltpu.async_copy(biota_v.at[r], tag_hbm.at[sidx_v.at[r]],
                                    sem2) for r in range(R)]
            for g in tsc:
                g.wait()
        plsc.subcore_barrier()
        tg = [pltpu.async_copy(tag_hbm.at[tidx_v.at[r]], tvals_v.at[r], sem2)
              for r in range(R)]
        for g in tg:
            g.wait()
        pltpu.sync_copy(tvals_v, t2_hbm.at[pl.ds(s * R, R)])
        tsg = [pltpu.async_copy(ts_hbm.at[tvals_v.at[r]], tsr_v.at[r], sem2)
               for r in range(R)]
        for g in tsg:
            g.wait()
        tss = [pltpu.async_copy(tsr_v.at[r], tim_ref.at[tidx_v.at[r]], sem2)
               for r in range(R)]
        for g in tss:
            g.wait()


_tag_and_times = pl.kernel(
    _tag_body,
    out_type=(
        jax.ShapeDtypeStruct((ROWS2, CH), jnp.int32),
        jax.ShapeDtypeStruct((N_NODES + B,), jnp.int32),
    ),
    mesh=_mesh(),
    scratch_types=[
        pltpu.VMEM((R, CH), jnp.int32),
        pltpu.VMEM((R, CH), jnp.int32),
        pltpu.VMEM((R, CH), jnp.int32),
        pltpu.VMEM((R, CH), jnp.int32),
        pltpu.VMEM((R, CH), jnp.float32),
        pltpu.SemaphoreType.DMA,
    ],
    compiler_params=pltpu.CompilerParams(use_tc_tiling_on_sc=False),
)


def _gather_body(mem_ref, ids_hbm, t2_hbm, h_hbm, idx_v, rows_v, sem):
    del t2_hbm
    c = lax.axis_index("c")
    s = lax.axis_index("s")
    wid = s * NC + c
    base = wid * BPW
    pltpu.sync_copy(ids_hbm.at[pl.ds(base, BPW)], idx_v)

    @pl.loop(0, BPW)
    def _fire(k):
        i = _scalar_at(idx_v, k)
        pltpu.async_copy(mem_ref.at[pl.ds(i, 1)], rows_v.at[pl.ds(k, 1)], sem)

        @pl.when(k >= KF)
        def _():
            pltpu.make_async_copy(mem_ref.at[pl.ds(0, 1)],
                                  rows_v.at[pl.ds(k - KF, 1)], sem).wait()

    @pl.loop(BPW - KF, BPW)
    def _drain(k):
        pltpu.make_async_copy(mem_ref.at[pl.ds(0, 1)],
                              rows_v.at[pl.ds(k, 1)], sem).wait()

    pltpu.sync_copy(rows_v, h_hbm.at[pl.ds(base, BPW)])


_gather = pl.kernel(
    _gather_body,
    out_type=jax.ShapeDtypeStruct((B, D), jnp.float32),
    mesh=_mesh(),
    scratch_types=[
        pltpu.VMEM((BPW,), jnp.int32),
        pltpu.VMEM((BPW, D), jnp.float32),
        pltpu.SemaphoreType.DMA,
    ],
    compiler_params=pltpu.CompilerParams(needs_layout_passes=False),
)


GRU_BLK = 1024


def _gru_body(x_ref, h_ref, wih_ref, whh_ref, bih_ref, bhh_ref, o_ref):
    x = x_ref[...]
    h = h_ref[...]
    gi = jnp.dot(x, wih_ref[...], preferred_element_type=jnp.float32) + bih_ref[...]
    gh = jnp.dot(h, whh_ref[...], preferred_element_type=jnp.float32) + bhh_ref[...]
    r = jax.nn.sigmoid(gi[:, :D] + gh[:, :D])
    z = jax.nn.sigmoid(gi[:, D:2 * D] + gh[:, D:2 * D])
    n = jnp.tanh(gi[:, 2 * D:] + r * gh[:, 2 * D:])
    o_ref[...] = (1.0 - z) * n + z * h


def _gru(msgs, h, w_ih_t, w_hh_t, b_ih2, b_hh2):
    return pl.pallas_call(
        _gru_body,
        grid=(B // GRU_BLK,),
        in_specs=[
            pl.BlockSpec((GRU_BLK, D), lambda i: (i, 0)),
            pl.BlockSpec((GRU_BLK, D), lambda i: (i, 0)),
            pl.BlockSpec((D, 3 * D), lambda i: (0, 0)),
            pl.BlockSpec((D, 3 * D), lambda i: (0, 0)),
            pl.BlockSpec((1, 3 * D), lambda i: (0, 0)),
            pl.BlockSpec((1, 3 * D), lambda i: (0, 0)),
        ],
        out_specs=pl.BlockSpec((GRU_BLK, D), lambda i: (i, 0)),
        out_shape=jax.ShapeDtypeStruct((B, D), jnp.float32),
    )(msgs, h, w_ih_t, w_hh_t, b_ih2, b_hh2)


def _scatter_body(newh_hbm, t2_hbm, ids_hbm, mem_ref,
                  idx_v, tw_v, rows_v, sem, sem2):
    c = lax.axis_index("c")
    s = lax.axis_index("s")
    wid = s * NC + c
    base = wid * BPW
    pltpu.sync_copy(ids_hbm.at[pl.ds(base, BPW)], idx_v)
    pltpu.sync_copy(t2_hbm.at[pl.ds(base, BPW)], tw_v)

    @pl.loop(0, BPW)
    def _fire_g(k):
        t = _scalar_at(tw_v, k)
        pltpu.async_copy(newh_hbm.at[pl.ds(t, 1)], rows_v.at[pl.ds(k, 1)], sem)

        @pl.when(k >= KF)
        def _():
            pltpu.make_async_copy(newh_hbm.at[pl.ds(0, 1)],
                                  rows_v.at[pl.ds(k - KF, 1)], sem).wait()

    @pl.loop(BPW - KF, BPW)
    def _drain_g(k):
        pltpu.make_async_copy(newh_hbm.at[pl.ds(0, 1)],
                              rows_v.at[pl.ds(k, 1)], sem).wait()

    @pl.loop(0, BPW)
    def _fire_s(k):
        i = _scalar_at(idx_v, k)
        pltpu.async_copy(rows_v.at[pl.ds(k, 1)], mem_ref.at[pl.ds(i, 1)], sem2)

        @pl.when(k >= KF)
        def _():
            pltpu.make_async_copy(rows_v.at[pl.ds(k - KF, 1)],
                                  mem_ref.at[pl.ds(0, 1)], sem2).wait()

    @pl.loop(BPW - KF, BPW)
    def _drain_s(k):
        pltpu.make_async_copy(rows_v.at[pl.ds(k, 1)],
                              mem_ref.at[pl.ds(0, 1)], sem2).wait()


_scatter = pl.kernel(
    _scatter_body,
    out_type=(),
    mesh=_mesh(),
    scratch_types=[
        pltpu.VMEM((BPW,), jnp.int32),
        pltpu.VMEM((BPW,), jnp.int32),
        pltpu.VMEM((BPW, D), jnp.float32),
        pltpu.SemaphoreType.DMA,
        pltpu.SemaphoreType.DMA,
    ],
    compiler_params=pltpu.CompilerParams(needs_layout_passes=False),
)


def kernel(node_memories, node_last_updated_times, unique_node_ids,
           unique_node_messages, unique_node_timestamps,
           W_ih, W_hh, b_ih, b_hh):
    ids2 = unique_node_ids.reshape(ROWS2, CH)
    biota = jnp.arange(B, dtype=jnp.int32).reshape(ROWS2, CH)
    mem_ref = jax.new_ref(node_memories)
    tim_ref = jax.new_ref(node_last_updated_times)
    t2, _tag = _tag_and_times(ids2, biota, unique_node_timestamps, tim_ref)
    h = _gather(mem_ref, unique_node_ids, t2)
    new_h = _gru(unique_node_messages, h, W_ih.T, W_hh.T,
                 b_ih.reshape(1, 3 * D), b_hh.reshape(1, 3 * D))
    _scatter(new_h, t2.reshape(B), unique_node_ids, mem_ref)
    return jax.freeze(mem_ref), jax.freeze(tim_ref)
